# Initial kernel scaffold; baseline (speedup 1.0000x reference)
import jax
import jax.numpy as jnp
from jax.experimental import pallas as pl


def kernel(X, edge_index, step_index, step_emb, W0_0, b0_0, W1_0, b1_0, Ws0, bs0, Ws1, bs1):
    raise NotImplementedError("write your pallas kernel here")


# speedup vs baseline: 29.5728x; 29.5728x over previous
import functools

import jax
import jax.numpy as jnp
from jax import lax
from jax.experimental import pallas as pl
from jax.experimental.pallas import tpu as pltpu
from jax.experimental.pallas import tpu_sc as plsc

N = 100000
E = 3200000
NPAD = 100352
NT = 32
EPT = 100352
EPAD = NT * EPT
C = 1024
NCHUNK = EPT // C
NODES_PER_TILE = NPAD // 16
SUB = 784
NSUBS = NODES_PER_TILE // SUB
D = 8
F32 = jnp.float32
I32 = jnp.int32


@functools.cache
def _mesh():
    return plsc.VectorSubcoreMesh(core_axis_name="c", subcore_axis_name="s")


_SC_PARAMS = pltpu.CompilerParams(use_tc_tiling_on_sc=False,
                                  needs_layout_passes=False)


def _layer_body(tin, ain0, ain1, dinv, rowp, colp, zeros8, consts,
                t_out, a_out,
                tbuf, a0b, a1b, dbuf, toutb, cbuf, ridx, cidx, msg, acc, sem):
    c = lax.axis_index("c")
    s = lax.axis_index("s")
    wid = c * 16 + s
    lane = lax.iota(I32, 16)
    cpar = lane & 1
    copp = 1 - cpar
    two = jnp.full((16,), 2, I32)

    pltpu.sync_copy(consts, cbuf)
    wA = cbuf[0, :]
    wB = cbuf[1, :]
    w1A = cbuf[2, :]
    w1B = cbuf[3, :]
    uv = cbuf[4, :]
    bv = cbuf[5, :] + cbuf[6, :]
    sel = cbuf[7, :]
    ra = cbuf[8, :]
    rb = cbuf[9, :]

    r0z = s * NODES_PER_TILE
    pltpu.sync_copy(zeros8.at[pl.ds(r0z, NODES_PER_TILE), :],
                    acc.at[pl.ds(r0z, NODES_PER_TILE), :])
    pltpu.sync_copy(zeros8.at[pl.ds(0, SUB), :], toutb)

    for j in range(NSUBS):
        r0 = s * NODES_PER_TILE + j * SUB
        pltpu.sync_copy(tin.at[c].at[pl.ds(r0, SUB), :], tbuf)
        pltpu.sync_copy(ain0.at[pl.ds(r0, SUB), :], a0b)
        pltpu.sync_copy(ain1.at[pl.ds(r0, SUB), :], a1b)
        pltpu.sync_copy(dinv.at[pl.ds(r0, SUB)], dbuf)

        def pair(g, carry):
            n = (g * 16 + lane) >> 1
            yv = plsc.load_gather(tbuf, [n, cpar])
            ys = plsc.load_gather(tbuf, [n, copp])
            a0v = plsc.load_gather(a0b, [n, cpar])
            a1v = plsc.load_gather(a1b, [n, cpar])
            a0s = plsc.load_gather(a0b, [n, copp])
            a1s = plsc.load_gather(a1b, [n, copp])
            s2 = (plsc.load_gather(tbuf, [n, two])
                  + plsc.load_gather(a0b, [n, two])
                  + plsc.load_gather(a1b, [n, two]))
            dv = plsc.load_gather(dbuf, [n])
            sm = yv + a0v + a1v
            sms = ys + a0s + a1s
            rdv = sel * (1.0 / dv - 1.0) + 1.0
            out = ((yv * wA + ys * wB) * rdv
                   + (sm * w1A + sms * w1B) * dv
                   + s2 * dv * uv + bv)
            val = (ra * jnp.maximum(out, 0.0) + rb * out) * dv
            plsc.store_scatter(toutb, [n, cpar], val)
            return carry

        lax.fori_loop(0, SUB * 2 // 16, pair, 0)

        def ch2w(g, carry):
            n = g * 16 + lane
            dv = dbuf[pl.ds(g * 16, 16)]
            plsc.store_scatter(toutb, [n, two], dv)
            return carry

        lax.fori_loop(0, SUB // 16, ch2w, 0)
        pltpu.sync_copy(toutb, t_out.at[c].at[pl.ds(r0, SUB), :])

    plsc.subcore_barrier()

    ebase = wid * EPT

    def chunk(k, carry):
        off = ebase + k * C
        pltpu.sync_copy(rowp.at[pl.ds(off, C)], ridx)
        pltpu.sync_copy(colp.at[pl.ds(off, C)], cidx)
        pltpu.async_copy(t_out.at[c].at[ridx], msg, sem).wait()
        pltpu.sync_copy(msg, acc.at[cidx], add=True)
        return carry

    lax.fori_loop(0, NCHUNK, chunk, 0)
    plsc.subcore_barrier()
    pltpu.sync_copy(acc.at[pl.ds(r0z, NODES_PER_TILE), :],
                    a_out.at[c].at[pl.ds(r0z, NODES_PER_TILE), :])


@functools.cache
def _get_k_layer():
    return pl.kernel(
        _layer_body,
        out_type=(
            jax.ShapeDtypeStruct((2, NPAD, D), F32),
            jax.ShapeDtypeStruct((2, NPAD, D), F32),
        ),
        mesh=_mesh(),
        compiler_params=_SC_PARAMS,
        scratch_types=[
            pltpu.VMEM((SUB, D), F32),
            pltpu.VMEM((SUB, D), F32),
            pltpu.VMEM((SUB, D), F32),
            pltpu.VMEM((SUB,), F32),
            pltpu.VMEM((SUB, D), F32),
            pltpu.VMEM((10, 16), F32),
            pltpu.VMEM((C,), I32),
            pltpu.VMEM((C,), I32),
            pltpu.VMEM((C, D), F32),
            pltpu.VMEM_SHARED((NPAD, D), F32),
            pltpu.SemaphoreType.DMA,
        ],
    )


def _pre_body(d0, d1, emb, w0, w1, b0, b1, si, dinv_out, uc_out):
    deg = d0[...] + d1[...] + 1.0
    dinv_out[...] = 1.0 / jnp.sqrt(deg)
    idx = si[0]
    rows = lax.broadcasted_iota(I32, (100, 8), 0)
    sel = jnp.where(rows == idx, emb[...], 0.0)
    s = jnp.sum(sel, axis=0)
    u0 = jnp.sum(s[:, None] * w0[...][2:, :], axis=0) + b0[...][0]
    u1 = jnp.sum(s[:, None] * w1[...][2:, :], axis=0) + b1[...][0]
    uc_out[...] = jnp.concatenate([u0, u1]).reshape(1, 4)


def _k_pre(d0, d1, emb, w0, w1, b0, b1, si):
    return pl.pallas_call(
        _pre_body,
        out_shape=[
            jax.ShapeDtypeStruct((784, 128), F32),
            jax.ShapeDtypeStruct((1, 4), F32),
        ],
        in_specs=[pl.BlockSpec(memory_space=pltpu.VMEM)] * 7
        + [pl.BlockSpec(memory_space=pltpu.SMEM)],
        out_specs=[
            pl.BlockSpec(memory_space=pltpu.VMEM),
            pl.BlockSpec(memory_space=pltpu.VMEM),
        ],
    )(d0, d1, emb, w0, w1, b0, b1, si)


def _post_body(y0, y1, a00, a01, a10, a11, dv, w0, w1, b0, b1,
               h0_out, h1_out):
    dvv = dv[...]
    h70 = y0[...] / dvv
    h71 = y1[...] / dvv
    ah0 = (y0[...] + a00[...] + a10[...]) * dvv
    ah1 = (y1[...] + a01[...] + a11[...]) * dvv
    w0m = w0[...]
    w1m = w1[...]
    bias0 = b0[...][0, 0] + b1[...][0, 0]
    bias1 = b0[...][0, 1] + b1[...][0, 1]
    o0 = (h70 * w0m[0, 0] + h71 * w0m[1, 0]
          + ah0 * w1m[0, 0] + ah1 * w1m[1, 0] + bias0)
    o1 = (h70 * w0m[0, 1] + h71 * w0m[1, 1]
          + ah0 * w1m[0, 1] + ah1 * w1m[1, 1] + bias1)
    h0_out[...] = jnp.maximum(o0, 0.0)
    h1_out[...] = jnp.maximum(o1, 0.0)


def _k_post(y0, y1, a00, a01, a10, a11, dv, w0, w1, b0, b1):
    return pl.pallas_call(
        _post_body,
        out_shape=[
            jax.ShapeDtypeStruct((784, 128), F32),
            jax.ShapeDtypeStruct((784, 128), F32),
        ],
        in_specs=[pl.BlockSpec(memory_space=pltpu.VMEM)] * 11,
        out_specs=[
            pl.BlockSpec(memory_space=pltpu.VMEM),
            pl.BlockSpec(memory_space=pltpu.VMEM),
        ],
    )(y0, y1, a00, a01, a10, a11, dv, w0, w1, b0, b1)


def _pat(v0, v1):
    e = (jnp.arange(16) % 2) == 0
    return jnp.where(e, v0, v1)


def _consts(wA0, wA1, wB0, wB1, w1A0, w1A1, w1B0, w1B1,
            uv0, uv1, bv00, bv01, bv10, bv11, sel, ra, rb):
    z = jnp.zeros((16,), F32)
    rows = [
        _pat(wA0, wA1), _pat(wB0, wB1),
        _pat(w1A0, w1A1), _pat(w1B0, w1B1),
        _pat(uv0, uv1), _pat(bv00, bv01), _pat(bv10, bv11),
        z + sel, z + ra, z + rb,
    ]
    return jnp.stack(rows).astype(F32)


def _consts_mat(w0, w1, b0, b1):
    return _consts(w0[0, 0], w0[1, 1], w0[1, 0], w0[0, 1],
                   w1[0, 0], w1[1, 1], w1[1, 0], w1[0, 1],
                   0.0, 0.0, b0[0], b0[1], b1[0], b1[1], 1.0, 1.0, 0.0)


def kernel(X, edge_index, step_index, step_emb, W0_0, b0_0, W1_0, b1_0,
           Ws0, bs0, Ws1, bs1):
    row = edge_index[0]
    col = edge_index[1]
    padi = jnp.full((EPAD - E,), N, I32)
    rowp = jnp.concatenate([row, padi])
    colp = jnp.concatenate([col, padi])
    zeros8 = jnp.zeros((NPAD, D), F32)
    zeros28 = jnp.zeros((2, NPAD, D), F32)
    ones_n = jnp.ones((NPAD,), F32)
    xt8 = jnp.zeros((NPAD, D), F32).at[:N, 0:2].set(X)
    xt28 = jnp.broadcast_to(xt8[None], (2, NPAD, D))

    klayer = _get_k_layer()
    zc = 0.0
    c_deg = _consts(zc, zc, zc, zc, zc, zc, zc, zc, zc, zc,
                    1.0, 1.0, zc, zc, 0.0, 1.0, 0.0)
    _, a_deg = klayer(zeros28, zeros8, zeros8, ones_n, rowp, colp, zeros8,
                      c_deg)

    d0p = a_deg[0, :, 0].reshape(784, 128)
    d1p = a_deg[1, :, 0].reshape(784, 128)
    dinv2d, uc = _k_pre(d0p, d1p, step_emb, W0_0, W1_0,
                        b0_0.reshape(1, 2), b1_0.reshape(1, 2),
                        step_index.reshape(1,))
    dinv = dinv2d.reshape(NPAD)

    c_build = _consts(1.0, 1.0, zc, zc, zc, zc, zc, zc,
                      zc, zc, zc, zc, zc, zc, 0.0, 0.0, 1.0)
    t, a = klayer(xt28, zeros8, zeros8, dinv, rowp, colp, zeros8, c_build)

    u = uc[0]
    w0h = W0_0[:2]
    w1h = W1_0[:2]
    c_l1 = _consts(w0h[0, 0], w0h[1, 1], w0h[1, 0], w0h[0, 1],
                   w1h[0, 0], w1h[1, 1], w1h[1, 0], w1h[0, 1],
                   u[2], u[3], u[0], u[1], zc, zc, 1.0, 1.0, 0.0)
    t, a = klayer(t, a[0], a[1], dinv, rowp, colp, zeros8, c_l1)

    for i in range(6):
        ci = _consts_mat(Ws0[i], Ws1[i], bs0[i], bs1[i])
        t, a = klayer(t, a[0], a[1], dinv, rowp, colp, zeros8, ci)

    y0 = t[0, :, 0].reshape(784, 128)
    y1 = t[0, :, 1].reshape(784, 128)
    a00 = a[0, :, 0].reshape(784, 128)
    a01 = a[0, :, 1].reshape(784, 128)
    a10 = a[1, :, 0].reshape(784, 128)
    a11 = a[1, :, 1].reshape(784, 128)
    h0p, h1p = _k_post(y0, y1, a00, a01, a10, a11, dinv2d,
                       Ws0[6], Ws1[6], bs0[6].reshape(1, 2),
                       bs1[6].reshape(1, 2))
    h = jnp.stack([h0p.reshape(NPAD), h1p.reshape(NPAD)], axis=1)
    return h[:N]


# speedup vs baseline: 41.1780x; 1.3924x over previous
import functools

import jax
import jax.numpy as jnp
from jax import lax
from jax.experimental import pallas as pl
from jax.experimental.pallas import tpu as pltpu
from jax.experimental.pallas import tpu_sc as plsc

N = 100000
E = 3200000
NPAD = 100352
NT = 32
EPT = 100352
EPAD = NT * EPT
C = 2048
NCHUNK = EPT // C
NODES_PER_TILE = NPAD // 16
SUB = 784
NSUBS = NODES_PER_TILE // SUB
D = 8
F32 = jnp.float32
I32 = jnp.int32


@functools.cache
def _mesh():
    return plsc.VectorSubcoreMesh(core_axis_name="c", subcore_axis_name="s")


_SC_PARAMS = pltpu.CompilerParams(use_tc_tiling_on_sc=False,
                                  needs_layout_passes=False)


def _layer_body(tin, ain0, ain1, dinv, rowp, colp, zeros8, consts,
                t_out, a_out,
                tbuf, a0b, a1b, dbuf, toutb, cbuf,
                ridx0, cidx0, msg0, ridx1, cidx1, msg1,
                acc, semg0, semg1, sems0, sems1):
    c = lax.axis_index("c")
    s = lax.axis_index("s")
    wid = c * 16 + s
    lane = lax.iota(I32, 16)
    cpar = lane & 1
    copp = 1 - cpar
    two = jnp.full((16,), 2, I32)

    pltpu.sync_copy(consts, cbuf)
    wA = cbuf[0, :]
    wB = cbuf[1, :]
    w1A = cbuf[2, :]
    w1B = cbuf[3, :]
    uv = cbuf[4, :]
    bv = cbuf[5, :] + cbuf[6, :]
    sel = cbuf[7, :]
    ra = cbuf[8, :]
    rb = cbuf[9, :]

    r0z = s * NODES_PER_TILE
    pltpu.sync_copy(zeros8.at[pl.ds(r0z, NODES_PER_TILE), :],
                    acc.at[pl.ds(r0z, NODES_PER_TILE), :])
    pltpu.sync_copy(zeros8.at[pl.ds(0, SUB), :], toutb)

    for j in range(NSUBS):
        r0 = s * NODES_PER_TILE + j * SUB
        pltpu.sync_copy(tin.at[c].at[pl.ds(r0, SUB), :], tbuf)
        pltpu.sync_copy(ain0.at[pl.ds(r0, SUB), :], a0b)
        pltpu.sync_copy(ain1.at[pl.ds(r0, SUB), :], a1b)
        pltpu.sync_copy(dinv.at[pl.ds(r0, SUB)], dbuf)

        def pair(g, carry):
            n = (g * 16 + lane) >> 1
            yv = plsc.load_gather(tbuf, [n, cpar])
            ys = plsc.load_gather(tbuf, [n, copp])
            a0v = plsc.load_gather(a0b, [n, cpar])
            a1v = plsc.load_gather(a1b, [n, cpar])
            a0s = plsc.load_gather(a0b, [n, copp])
            a1s = plsc.load_gather(a1b, [n, copp])
            s2 = (plsc.load_gather(tbuf, [n, two])
                  + plsc.load_gather(a0b, [n, two])
                  + plsc.load_gather(a1b, [n, two]))
            dv = plsc.load_gather(dbuf, [n])
            sm = yv + a0v + a1v
            sms = ys + a0s + a1s
            rdv = sel * (1.0 / dv - 1.0) + 1.0
            out = ((yv * wA + ys * wB) * rdv
                   + (sm * w1A + sms * w1B) * dv
                   + s2 * dv * uv + bv)
            val = (ra * jnp.maximum(out, 0.0) + rb * out) * dv
            plsc.store_scatter(toutb, [n, cpar], val)
            return carry

        lax.fori_loop(0, SUB * 2 // 16, pair, 0)

        def ch2w(g, carry):
            n = g * 16 + lane
            dv = dbuf[pl.ds(g * 16, 16)]
            plsc.store_scatter(toutb, [n, two], dv)
            return carry

        lax.fori_loop(0, SUB // 16, ch2w, 0)
        pltpu.sync_copy(toutb, t_out.at[c].at[pl.ds(r0, SUB), :])

    plsc.subcore_barrier()

    ebase = wid * EPT
    bufs = [(ridx0, cidx0, msg0, semg0, sems0),
            (ridx1, cidx1, msg1, semg1, sems1)]

    def load_and_gather(k, b):
        ridx, cidx, msg, semg, _ = bufs[b]
        off = ebase + k * C
        pltpu.sync_copy(rowp.at[pl.ds(off, C)], ridx)
        pltpu.sync_copy(colp.at[pl.ds(off, C)], cidx)
        return pltpu.async_copy(t_out.at[c].at[ridx], msg, semg)

    gd = [None, None]
    sd = [None, None]
    gd[0] = load_and_gather(0, 0)
    for k in range(NCHUNK):
        b = k & 1
        nb = 1 - b
        if k + 1 < NCHUNK:
            if sd[nb] is not None:
                sd[nb].wait()
                sd[nb] = None
            gd[nb] = load_and_gather(k + 1, nb)
        gd[b].wait()
        ridx, cidx, msg, _, sems = bufs[b]
        sd[b] = pltpu.async_copy(msg, acc.at[cidx], sems, add=True)
    for b in range(2):
        if sd[b] is not None:
            sd[b].wait()
    plsc.subcore_barrier()
    pltpu.sync_copy(acc.at[pl.ds(r0z, NODES_PER_TILE), :],
                    a_out.at[c].at[pl.ds(r0z, NODES_PER_TILE), :])


@functools.cache
def _get_k_layer():
    return pl.kernel(
        _layer_body,
        out_type=(
            jax.ShapeDtypeStruct((2, NPAD, D), F32),
            jax.ShapeDtypeStruct((2, NPAD, D), F32),
        ),
        mesh=_mesh(),
        compiler_params=_SC_PARAMS,
        scratch_types=[
            pltpu.VMEM((SUB, D), F32),
            pltpu.VMEM((SUB, D), F32),
            pltpu.VMEM((SUB, D), F32),
            pltpu.VMEM((SUB,), F32),
            pltpu.VMEM((SUB, D), F32),
            pltpu.VMEM((10, 16), F32),
            pltpu.VMEM((C,), I32),
            pltpu.VMEM((C,), I32),
            pltpu.VMEM((C, D), F32),
            pltpu.VMEM((C,), I32),
            pltpu.VMEM((C,), I32),
            pltpu.VMEM((C, D), F32),
            pltpu.VMEM_SHARED((NPAD, D), F32),
            pltpu.SemaphoreType.DMA,
            pltpu.SemaphoreType.DMA,
            pltpu.SemaphoreType.DMA,
            pltpu.SemaphoreType.DMA,
        ],
    )


def _pre_body(d0, d1, emb, w0, w1, b0, b1, si, dinv_out, uc_out):
    deg = d0[...] + d1[...] + 1.0
    dinv_out[...] = 1.0 / jnp.sqrt(deg)
    idx = si[0]
    rows = lax.broadcasted_iota(I32, (100, 8), 0)
    sel = jnp.where(rows == idx, emb[...], 0.0)
    s = jnp.sum(sel, axis=0)
    u0 = jnp.sum(s[:, None] * w0[...][2:, :], axis=0) + b0[...][0]
    u1 = jnp.sum(s[:, None] * w1[...][2:, :], axis=0) + b1[...][0]
    uc_out[...] = jnp.concatenate([u0, u1]).reshape(1, 4)


def _k_pre(d0, d1, emb, w0, w1, b0, b1, si):
    return pl.pallas_call(
        _pre_body,
        out_shape=[
            jax.ShapeDtypeStruct((784, 128), F32),
            jax.ShapeDtypeStruct((1, 4), F32),
        ],
        in_specs=[pl.BlockSpec(memory_space=pltpu.VMEM)] * 7
        + [pl.BlockSpec(memory_space=pltpu.SMEM)],
        out_specs=[
            pl.BlockSpec(memory_space=pltpu.VMEM),
            pl.BlockSpec(memory_space=pltpu.VMEM),
        ],
    )(d0, d1, emb, w0, w1, b0, b1, si)


def _post_body(y0, y1, a00, a01, a10, a11, dv, w0, w1, b0, b1,
               h0_out, h1_out):
    dvv = dv[...]
    h70 = y0[...] / dvv
    h71 = y1[...] / dvv
    ah0 = (y0[...] + a00[...] + a10[...]) * dvv
    ah1 = (y1[...] + a01[...] + a11[...]) * dvv
    w0m = w0[...]
    w1m = w1[...]
    bias0 = b0[...][0, 0] + b1[...][0, 0]
    bias1 = b0[...][0, 1] + b1[...][0, 1]
    o0 = (h70 * w0m[0, 0] + h71 * w0m[1, 0]
          + ah0 * w1m[0, 0] + ah1 * w1m[1, 0] + bias0)
    o1 = (h70 * w0m[0, 1] + h71 * w0m[1, 1]
          + ah0 * w1m[0, 1] + ah1 * w1m[1, 1] + bias1)
    h0_out[...] = jnp.maximum(o0, 0.0)
    h1_out[...] = jnp.maximum(o1, 0.0)


def _k_post(y0, y1, a00, a01, a10, a11, dv, w0, w1, b0, b1):
    return pl.pallas_call(
        _post_body,
        out_shape=[
            jax.ShapeDtypeStruct((784, 128), F32),
            jax.ShapeDtypeStruct((784, 128), F32),
        ],
        in_specs=[pl.BlockSpec(memory_space=pltpu.VMEM)] * 11,
        out_specs=[
            pl.BlockSpec(memory_space=pltpu.VMEM),
            pl.BlockSpec(memory_space=pltpu.VMEM),
        ],
    )(y0, y1, a00, a01, a10, a11, dv, w0, w1, b0, b1)


def _pat(v0, v1):
    e = (jnp.arange(16) % 2) == 0
    return jnp.where(e, v0, v1)


def _consts(wA0, wA1, wB0, wB1, w1A0, w1A1, w1B0, w1B1,
            uv0, uv1, bv00, bv01, bv10, bv11, sel, ra, rb):
    z = jnp.zeros((16,), F32)
    rows = [
        _pat(wA0, wA1), _pat(wB0, wB1),
        _pat(w1A0, w1A1), _pat(w1B0, w1B1),
        _pat(uv0, uv1), _pat(bv00, bv01), _pat(bv10, bv11),
        z + sel, z + ra, z + rb,
    ]
    return jnp.stack(rows).astype(F32)


def _consts_mat(w0, w1, b0, b1):
    return _consts(w0[0, 0], w0[1, 1], w0[1, 0], w0[0, 1],
                   w1[0, 0], w1[1, 1], w1[1, 0], w1[0, 1],
                   0.0, 0.0, b0[0], b0[1], b1[0], b1[1], 1.0, 1.0, 0.0)


def kernel(X, edge_index, step_index, step_emb, W0_0, b0_0, W1_0, b1_0,
           Ws0, bs0, Ws1, bs1):
    row = edge_index[0]
    col = edge_index[1]
    padi = jnp.full((EPAD - E,), N, I32)
    rowp = jnp.concatenate([row, padi])
    colp = jnp.concatenate([col, padi])
    zeros8 = jnp.zeros((NPAD, D), F32)
    zeros28 = jnp.zeros((2, NPAD, D), F32)
    ones_n = jnp.ones((NPAD,), F32)
    xt8 = jnp.zeros((NPAD, D), F32).at[:N, 0:2].set(X)
    xt28 = jnp.broadcast_to(xt8[None], (2, NPAD, D))

    klayer = _get_k_layer()
    zc = 0.0
    c_deg = _consts(zc, zc, zc, zc, zc, zc, zc, zc, zc, zc,
                    1.0, 1.0, zc, zc, 0.0, 1.0, 0.0)
    _, a_deg = klayer(zeros28, zeros8, zeros8, ones_n, rowp, colp, zeros8,
                      c_deg)

    d0p = a_deg[0, :, 0].reshape(784, 128)
    d1p = a_deg[1, :, 0].reshape(784, 128)
    dinv2d, uc = _k_pre(d0p, d1p, step_emb, W0_0, W1_0,
                        b0_0.reshape(1, 2), b1_0.reshape(1, 2),
                        step_index.reshape(1,))
    dinv = dinv2d.reshape(NPAD)

    c_build = _consts(1.0, 1.0, zc, zc, zc, zc, zc, zc,
                      zc, zc, zc, zc, zc, zc, 0.0, 0.0, 1.0)
    t, a = klayer(xt28, zeros8, zeros8, dinv, rowp, colp, zeros8, c_build)

    u = uc[0]
    w0h = W0_0[:2]
    w1h = W1_0[:2]
    c_l1 = _consts(w0h[0, 0], w0h[1, 1], w0h[1, 0], w0h[0, 1],
                   w1h[0, 0], w1h[1, 1], w1h[1, 0], w1h[0, 1],
                   u[2], u[3], u[0], u[1], zc, zc, 1.0, 1.0, 0.0)
    t, a = klayer(t, a[0], a[1], dinv, rowp, colp, zeros8, c_l1)

    for i in range(6):
        ci = _consts_mat(Ws0[i], Ws1[i], bs0[i], bs1[i])
        t, a = klayer(t, a[0], a[1], dinv, rowp, colp, zeros8, ci)

    y0 = t[0, :, 0].reshape(784, 128)
    y1 = t[0, :, 1].reshape(784, 128)
    a00 = a[0, :, 0].reshape(784, 128)
    a01 = a[0, :, 1].reshape(784, 128)
    a10 = a[1, :, 0].reshape(784, 128)
    a11 = a[1, :, 1].reshape(784, 128)
    h0p, h1p = _k_post(y0, y1, a00, a01, a10, a11, dinv2d,
                       Ws0[6], Ws1[6], bs0[6].reshape(1, 2),
                       bs1[6].reshape(1, 2))
    h = jnp.stack([h0p.reshape(NPAD), h1p.reshape(NPAD)], axis=1)
    return h[:N]
